# baseline (speedup 1.0000x reference)
import functools

import jax
import jax.numpy as jnp
from jax import lax
from jax.experimental import pallas as pl
from jax.experimental.pallas import tpu as pltpu
from jax.experimental.pallas import tpu_sc as plsc

INDEX_DIM = 8192
NUM_CHANNELS = 128
SEQ_LEN = 8192
OUT_BATCH = 4

NUM_CORES = 2
NUM_SUBCORES = 16
NUM_WORKERS = NUM_CORES * NUM_SUBCORES
ROWS_PER_WORKER = SEQ_LEN // NUM_WORKERS
IDX_CHUNK = 128
CHUNKS = ROWS_PER_WORKER // IDX_CHUNK


@functools.partial(
    pl.kernel,
    mesh=plsc.VectorSubcoreMesh(core_axis_name="c", subcore_axis_name="s"),
    out_type=jax.ShapeDtypeStruct((OUT_BATCH, SEQ_LEN, NUM_CHANNELS), jnp.float32),
    scratch_types=[
        pltpu.VMEM((CHUNKS, IDX_CHUNK), jnp.int32),
        pltpu.VMEM((ROWS_PER_WORKER, NUM_CHANNELS), jnp.float32),
        pltpu.SemaphoreType.DMA,
        pltpu.SemaphoreType.DMA,
        pltpu.SemaphoreType.DMA,
    ],
)
def _embed_bcast(ids_hbm, table_hbm, out_hbm, idx_v, rows_v, isem, gsem, wsem):
    wid = lax.axis_index("c") * NUM_SUBCORES + lax.axis_index("s")
    base = wid * ROWS_PER_WORKER

    id_copies = [
        pltpu.async_copy(ids_hbm.at[wid * CHUNKS + c], idx_v.at[c], isem)
        for c in range(CHUNKS)
    ]
    gathers = []
    for c in range(CHUNKS):
        id_copies[c].wait()
        gathers.append(
            pltpu.async_copy(
                table_hbm.at[idx_v.at[c]],
                rows_v.at[pl.ds(c * IDX_CHUNK, IDX_CHUNK)],
                gsem,
            )
        )
    writes = []
    for c in range(CHUNKS):
        gathers[c].wait()
        chunk = rows_v.at[pl.ds(c * IDX_CHUNK, IDX_CHUNK)]
        writes += [
            pltpu.async_copy(
                chunk, out_hbm.at[b, pl.ds(base + c * IDX_CHUNK, IDX_CHUNK)], wsem
            )
            for b in range(OUT_BATCH)
        ]
    for w in writes:
        w.wait()


def kernel(batch_size, position_ids, position_embeddings):
    del batch_size
    ids2d = position_ids.reshape(SEQ_LEN // IDX_CHUNK, IDX_CHUNK)
    return _embed_bcast(ids2d, position_embeddings)
